# baseline (speedup 1.0000x reference)
import functools
import math

import jax
import jax.numpy as jnp
from jax import lax
from jax.experimental import pallas as pl
from jax.experimental.pallas import tpu as pltpu
from jax.experimental.pallas import tpu_sc as plsc

NC = 2
NS = 16
LANES = 16
CHUNK = 128
GSZ = 32
ZROWS = 196
NBUF = 8
PDEPTH = 4
BLK = 1024


def _geometry(n, e):
    npad = math.ceil((n + 1) / BLK) * BLK
    assert npad % NS == 0
    rows_per_tile = npad // NS
    assert rows_per_tile % ZROWS == 0
    cpt = math.ceil(e / (CHUNK * NS))
    cpt = math.ceil(cpt / (2 * GSZ)) * (2 * GSZ)
    e_pad = cpt * CHUNK * NS
    return npad, rows_per_tile, cpt, e_pad


def _make_sc_layer(split_features, npad, rows_per_tile, cpt):
    mesh = plsc.VectorSubcoreMesh(
        core_axis_name="c", subcore_axis_name="s", num_cores=NC,
        num_subcores=NS)
    out_type = [jax.ShapeDtypeStruct((NC, npad, LANES), jnp.float32),
                jax.ShapeDtypeStruct((NC, npad, LANES), jnp.float32)]
    scratch = [
        pltpu.VMEM((GSZ, CHUNK), jnp.int32),
        pltpu.VMEM((GSZ, CHUNK), jnp.int32),
        pltpu.VMEM((NBUF, CHUNK, LANES), jnp.float32),
        pltpu.VMEM((ZROWS, LANES), jnp.float32),
        pltpu.VMEM_SHARED((npad, LANES), jnp.float32),
        pltpu.SemaphoreType.DMA((NBUF,)),
        pltpu.SemaphoreType.DMA((NBUF,)),
    ]

    def body(*refs):
        if split_features:
            (tab_lo, tab_hi, ts_src, ts_dst, is_src, is_dst,
             out_t, out_i, src_v, dst_v, rows_v, zero_v, acc,
             gsem, ssem) = refs
        else:
            (tab, ts_src, ts_dst, is_src, is_dst,
             out_t, out_i, src_v, dst_v, rows_v, zero_v, acc,
             gsem, ssem) = refs
        cid = lax.axis_index("c")
        sid = lax.axis_index("s")

        def fill_zeros(i, carry):
            zero_v[i, :] = jnp.zeros((LANES,), jnp.float32)
            return carry
        lax.fori_loop(0, ZROWS, fill_zeros, 0)

        def run(src_hbm, dst_hbm, table, out_ref, ngroups, base_chunks):
            def clear(i, carry):
                pltpu.sync_copy(
                    zero_v,
                    acc.at[pl.ds(sid * rows_per_tile + i * ZROWS, ZROWS)])
                return carry
            lax.fori_loop(0, rows_per_tile // ZROWS, clear, 0)
            plsc.subcore_barrier()

            def group(g, carry):
                off = base_chunks + g * GSZ
                pltpu.sync_copy(src_hbm.at[pl.ds(off, GSZ)], src_v)
                pltpu.sync_copy(dst_hbm.at[pl.ds(off, GSZ)], dst_v)

                def step(j, c2):
                    pltpu.async_copy(table.at[src_v.at[j]], rows_v.at[0],
                                     gsem.at[0])
                    pltpu.make_async_copy(
                        table.at[src_v.at[j]], rows_v.at[0],
                        gsem.at[0]).wait()
                    pltpu.async_copy(rows_v.at[0], acc.at[dst_v.at[j]],
                                     ssem.at[0], add=True)
                    pltpu.make_async_copy(
                        rows_v.at[0], acc.at[dst_v.at[j]],
                        ssem.at[0]).wait()
                    return c2
                lax.fori_loop(0, GSZ, step, 0)
                return carry
            lax.fori_loop(0, ngroups, group, 0)
            plsc.subcore_barrier()
            pltpu.sync_copy(
                acc.at[pl.ds(sid * rows_per_tile, rows_per_tile)],
                out_ref.at[cid, pl.ds(sid * rows_per_tile, rows_per_tile)])
            plsc.subcore_barrier()

        if split_features:
            ngroups = cpt // GSZ
            base = sid * cpt

            @pl.when(cid == 0)
            def _lo():
                run(ts_src, ts_dst, tab_lo, out_t, ngroups, base)
                run(is_src, is_dst, tab_lo, out_i, ngroups, base)

            @pl.when(cid == 1)
            def _hi():
                run(ts_src, ts_dst, tab_hi, out_t, ngroups, base)
                run(is_src, is_dst, tab_hi, out_i, ngroups, base)
        else:
            wpt = cpt // 2
            ngroups = wpt // GSZ
            base = (sid * NC + cid) * wpt
            run(ts_src, ts_dst, tab, out_t, ngroups, base)
            run(is_src, is_dst, tab, out_i, ngroups, base)

    return pl.kernel(body, out_type=out_type, mesh=mesh,
                     scratch_types=scratch,
                     compiler_params=pltpu.CompilerParams(
                         use_tc_tiling_on_sc=False))


def _tc_layer0(St, Si, Wt0p, Wi0p, bt0, bi0, npad):
    grid = npad // BLK

    def body(st_ref, si_ref, wt_ref, wi_ref, bt_ref, bi_ref,
             hlo_ref, hhi_ref, aux_ref):
        st = st_ref[0] + st_ref[1]
        si = si_ref[0] + si_ref[1]
        ct = st[:, 6:7]
        ci = si[:, 6:7]
        ri = 1.0 / jnp.maximum(ci, 1.0)
        zi = ci * ri
        at = jnp.dot(st, wt_ref[...], preferred_element_type=jnp.float32)
        ai = ri * jnp.dot(si, wi_ref[...], preferred_element_type=jnp.float32)
        h = jax.nn.relu(at + ct * bt_ref[...] + ai + zi * bi_ref[...])
        hlo_ref[...] = h[:, :16]
        hhi_ref[...] = h[:, 16:]
        aux_ref[...] = jnp.concatenate(
            [ct, ri, zi, jnp.zeros((BLK, 5), jnp.float32)], axis=1)

    full = pl.BlockSpec((NC, BLK, LANES), lambda i: (0, i, 0))
    half = pl.BlockSpec((BLK, LANES), lambda i: (i, 0))
    w16 = pl.BlockSpec((16, 32), lambda i: (0, 0))
    b32 = pl.BlockSpec((1, 32), lambda i: (0, 0))
    return pl.pallas_call(
        body,
        grid=(grid,),
        in_specs=[full, full, w16, w16, b32, b32],
        out_specs=[half, half, pl.BlockSpec((BLK, 8), lambda i: (i, 0))],
        out_shape=[jax.ShapeDtypeStruct((npad, LANES), jnp.float32),
                   jax.ShapeDtypeStruct((npad, LANES), jnp.float32),
                   jax.ShapeDtypeStruct((npad, 8), jnp.float32)],
    )(St, Si, Wt0p, Wi0p, bt0, bi0)


def _tc_mid(St, Si, hlo, hhi, aux, Wt, bt, Wi, bi, npad):
    grid = npad // BLK

    def body(st_ref, si_ref, hlo_ref, hhi_ref, aux_ref,
             wt_ref, bt_ref, wi_ref, bi_ref, olo_ref, ohi_ref):
        wt = wt_ref[...]
        wi = wi_ref[...]
        at = (jnp.dot(st_ref[0], wt[:16], preferred_element_type=jnp.float32)
              + jnp.dot(st_ref[1], wt[16:], preferred_element_type=jnp.float32))
        al = (jnp.dot(si_ref[0], wi[:16], preferred_element_type=jnp.float32)
              + jnp.dot(si_ref[1], wi[16:], preferred_element_type=jnp.float32))
        ct = aux_ref[:, 0:1]
        ri = aux_ref[:, 1:2]
        zi = aux_ref[:, 2:3]
        conv = at + ct * bt_ref[...] + ri * al + zi * bi_ref[...]
        h = jax.nn.relu(conv)
        olo_ref[...] = h[:, :16] + hlo_ref[...]
        ohi_ref[...] = h[:, 16:] + hhi_ref[...]

    full = pl.BlockSpec((NC, BLK, LANES), lambda i: (0, i, 0))
    half = pl.BlockSpec((BLK, LANES), lambda i: (i, 0))
    aux_s = pl.BlockSpec((BLK, 8), lambda i: (i, 0))
    w32 = pl.BlockSpec((32, 32), lambda i: (0, 0))
    b32 = pl.BlockSpec((1, 32), lambda i: (0, 0))
    return pl.pallas_call(
        body,
        grid=(grid,),
        in_specs=[full, full, half, half, aux_s, w32, b32, w32, b32],
        out_specs=[half, half],
        out_shape=[jax.ShapeDtypeStruct((npad, LANES), jnp.float32),
                   jax.ShapeDtypeStruct((npad, LANES), jnp.float32)],
    )(St, Si, hlo, hhi, aux, Wt, bt, Wi, bi)


def _tc_final(St, Si, hlo, hhi, aux, Wt4, bt4, Wi4, bi4, Wproj, Wm, bm, npad):
    grid = npad // BLK

    def body(st_ref, si_ref, hlo_ref, hhi_ref, aux_ref, wt_ref, bt_ref,
             wi_ref, bi_ref, wp_ref, wm_ref, bm_ref, out_ref):
        wt = wt_ref[...]
        wi = wi_ref[...]
        wp = wp_ref[...]
        at = (jnp.dot(st_ref[0], wt[:16], preferred_element_type=jnp.float32)
              + jnp.dot(st_ref[1], wt[16:], preferred_element_type=jnp.float32))
        al = (jnp.dot(si_ref[0], wi[:16], preferred_element_type=jnp.float32)
              + jnp.dot(si_ref[1], wi[16:], preferred_element_type=jnp.float32))
        ct = aux_ref[:, 0:1]
        ri = aux_ref[:, 1:2]
        zi = aux_ref[:, 2:3]
        conv = at + ct * bt_ref[...] + ri * al + zi * bi_ref[...]
        z = jax.nn.relu(conv)
        skip = (jnp.dot(hlo_ref[...], wp[:16], preferred_element_type=jnp.float32)
                + jnp.dot(hhi_ref[...], wp[16:], preferred_element_type=jnp.float32))
        h4 = z + skip
        out_ref[...] = (jnp.dot(h4, wm_ref[...],
                                preferred_element_type=jnp.float32)
                        + bm_ref[...])

    full = pl.BlockSpec((NC, BLK, LANES), lambda i: (0, i, 0))
    half = pl.BlockSpec((BLK, LANES), lambda i: (i, 0))
    aux_s = pl.BlockSpec((BLK, 8), lambda i: (i, 0))
    w64 = pl.BlockSpec((32, 64), lambda i: (0, 0))
    b64 = pl.BlockSpec((1, 64), lambda i: (0, 0))
    wm_s = pl.BlockSpec((64, 32), lambda i: (0, 0))
    b32 = pl.BlockSpec((1, 32), lambda i: (0, 0))
    return pl.pallas_call(
        body,
        grid=(grid,),
        in_specs=[full, full, half, half, aux_s, w64, b64, w64, b64,
                  w64, wm_s, b32],
        out_specs=pl.BlockSpec((BLK, 32), lambda i: (i, 0)),
        out_shape=jax.ShapeDtypeStruct((npad, 32), jnp.float32),
    )(St, Si, hlo, hhi, aux, Wt4, bt4, Wi4, bi4, Wproj, Wm, bm)


def kernel(x, Wt0, bt0, Wi0, bi0, Wt1, bt1, Wi1, bi1, Wt2, bt2, Wi2, bi2,
           Wt3, bt3, Wi3, bi3, Wt4, bt4, Wi4, bi4, Wproj, Wm, bm,
           edge_index_temp, edge_index_intersects):
    n = x.shape[0]
    e = edge_index_temp.shape[1]
    npad, rows_per_tile, cpt, e_pad = _geometry(n, e)
    dump = n

    def prep(ei):
        src = jnp.concatenate(
            [ei[0], jnp.zeros((e_pad - e,), jnp.int32)])
        dst = jnp.concatenate(
            [ei[1], jnp.full((e_pad - e,), dump, jnp.int32)])
        return (src.reshape(e_pad // CHUNK, CHUNK),
                dst.reshape(e_pad // CHUNK, CHUNK))

    ts_src, ts_dst = prep(edge_index_temp)
    is_src, is_dst = prep(edge_index_intersects)

    x_aug = jnp.zeros((npad, LANES), jnp.float32)
    x_aug = x_aug.at[:n, :6].set(x)
    x_aug = x_aug.at[:n, 6].set(1.0)

    Wt0p = jnp.zeros((LANES, 32), jnp.float32).at[:6].set(Wt0)
    Wi0p = jnp.zeros((LANES, 32), jnp.float32).at[:6].set(Wi0)

    sc_l0 = _make_sc_layer(False, npad, rows_per_tile, cpt)
    sc_gen = _make_sc_layer(True, npad, rows_per_tile, cpt)

    St0, Si0 = sc_l0(x_aug, ts_src, ts_dst, is_src, is_dst)
    hlo, hhi, aux = _tc_layer0(St0, Si0, Wt0p, Wi0p,
                               bt0.reshape(1, -1), bi0.reshape(1, -1), npad)

    for Wt, bt, Wi, bi in ((Wt1, bt1, Wi1, bi1), (Wt2, bt2, Wi2, bi2),
                           (Wt3, bt3, Wi3, bi3)):
        St, Si = sc_gen(hlo, hhi, ts_src, ts_dst, is_src, is_dst)
        hlo, hhi = _tc_mid(St, Si, hlo, hhi, aux, Wt, bt.reshape(1, -1),
                           Wi, bi.reshape(1, -1), npad)

    St4, Si4 = sc_gen(hlo, hhi, ts_src, ts_dst, is_src, is_dst)
    out = _tc_final(St4, Si4, hlo, hhi, aux, Wt4, bt4.reshape(1, -1),
                    Wi4, bi4.reshape(1, -1), Wproj, Wm, bm.reshape(1, -1),
                    npad)
    return out[:n]


# speedup vs baseline: 2.7489x; 2.7489x over previous
import functools
import math

import jax
import jax.numpy as jnp
from jax import lax
from jax.experimental import pallas as pl
from jax.experimental.pallas import tpu as pltpu
from jax.experimental.pallas import tpu_sc as plsc

NC = 2
NS = 16
LANES = 16
CHUNK = 128
GSZ = 4
NBUF = 2
ZROWS = 196
BLK = 1024


def _geometry(n, e):
    npad = math.ceil((n + 1) / BLK) * BLK
    assert npad % NS == 0
    rows_per_tile = npad // NS
    assert rows_per_tile % ZROWS == 0
    cpt = math.ceil(e / (CHUNK * NS))
    cpt = math.ceil(cpt / (2 * GSZ * NBUF)) * (2 * GSZ * NBUF)
    e_pad = cpt * CHUNK * NS
    return npad, rows_per_tile, cpt, e_pad


def _make_sc_layer(split_features, npad, rows_per_tile, cpt):
    mesh = plsc.VectorSubcoreMesh(
        core_axis_name="c", subcore_axis_name="s", num_cores=NC,
        num_subcores=NS)
    out_type = [jax.ShapeDtypeStruct((NC, npad, LANES), jnp.float32),
                jax.ShapeDtypeStruct((NC, npad, LANES), jnp.float32)]
    scratch = [
        pltpu.VMEM((NBUF, GSZ * CHUNK), jnp.int32),
        pltpu.VMEM((NBUF, GSZ * CHUNK), jnp.int32),
        pltpu.VMEM((NBUF, GSZ * CHUNK, LANES), jnp.float32),
        pltpu.VMEM((ZROWS, LANES), jnp.float32),
        pltpu.VMEM_SHARED((npad, LANES), jnp.float32),
        pltpu.SemaphoreType.DMA,
        pltpu.SemaphoreType.DMA,
        pltpu.SemaphoreType.DMA,
        pltpu.SemaphoreType.DMA,
    ]

    def body(*refs):
        if split_features:
            (tab_lo, tab_hi, ts_src, ts_dst, is_src, is_dst,
             out_t, out_i, src_v, dst_v, rows_v, zero_v, acc,
             gsem0, gsem1, ssem0, ssem1) = refs
        else:
            (tab, ts_src, ts_dst, is_src, is_dst,
             out_t, out_i, src_v, dst_v, rows_v, zero_v, acc,
             gsem0, gsem1, ssem0, ssem1) = refs
        gsems = (gsem0, gsem1)
        ssems = (ssem0, ssem1)
        cid = lax.axis_index("c")
        sid = lax.axis_index("s")

        def fill_zeros(i, carry):
            zero_v[i, :] = jnp.zeros((LANES,), jnp.float32)
            return carry
        lax.fori_loop(0, ZROWS, fill_zeros, 0)

        def run(src_hbm, dst_hbm, table, out_ref, ngroups, base_groups):
            def clear(i, carry):
                pltpu.sync_copy(
                    zero_v,
                    acc.at[pl.ds(sid * rows_per_tile + i * ZROWS, ZROWS)])
                return carry
            lax.fori_loop(0, rows_per_tile // ZROWS, clear, 0)
            plsc.subcore_barrier()

            def idx_load(b, g):
                pltpu.sync_copy(src_hbm.at[base_groups + g], src_v.at[b])
                pltpu.sync_copy(dst_hbm.at[base_groups + g], dst_v.at[b])

            def gath(b):
                return pltpu.make_async_copy(
                    table.at[src_v.at[b]], rows_v.at[b], gsems[b])

            def scat(b):
                return pltpu.make_async_copy(
                    rows_v.at[b], acc.at[dst_v.at[b]], ssems[b])

            idx_load(0, 0)
            gath(0).start()

            def step(i, carry):
                for b in range(NBUF):
                    g = i * NBUF + b
                    nb = (b + 1) % NBUF

                    @pl.when(g + 1 < ngroups)
                    def _start_next():
                        @pl.when(g >= 1)
                        def _drain_prev():
                            scat(nb).wait()
                        idx_load(nb, g + 1)
                        gath(nb).start()

                    gath(b).wait()
                    scat(b).start(add=True)
                return carry
            lax.fori_loop(0, ngroups // NBUF, step, 0)
            scat(0).wait()
            scat(1).wait()
            plsc.subcore_barrier()
            pltpu.sync_copy(
                acc.at[pl.ds(sid * rows_per_tile, rows_per_tile)],
                out_ref.at[cid, pl.ds(sid * rows_per_tile, rows_per_tile)])
            plsc.subcore_barrier()

        if split_features:
            ngroups = cpt // GSZ
            base = sid * ngroups

            @pl.when(cid == 0)
            def _lo():
                run(ts_src, ts_dst, tab_lo, out_t, ngroups, base)
                run(is_src, is_dst, tab_lo, out_i, ngroups, base)

            @pl.when(cid == 1)
            def _hi():
                run(ts_src, ts_dst, tab_hi, out_t, ngroups, base)
                run(is_src, is_dst, tab_hi, out_i, ngroups, base)
        else:
            ngroups = cpt // (2 * GSZ)
            base = (sid * NC + cid) * ngroups
            run(ts_src, ts_dst, tab, out_t, ngroups, base)
            run(is_src, is_dst, tab, out_i, ngroups, base)

    return pl.kernel(body, out_type=out_type, mesh=mesh,
                     scratch_types=scratch,
                     compiler_params=pltpu.CompilerParams(
                         use_tc_tiling_on_sc=False))


def _tc_layer0(St, Si, Wt0p, Wi0p, bt0, bi0, npad):
    grid = npad // BLK

    def body(st_ref, si_ref, wt_ref, wi_ref, bt_ref, bi_ref,
             hlo_ref, hhi_ref, aux_ref):
        st = st_ref[0] + st_ref[1]
        si = si_ref[0] + si_ref[1]
        ct = st[:, 6:7]
        ci = si[:, 6:7]
        ri = 1.0 / jnp.maximum(ci, 1.0)
        zi = ci * ri
        at = jnp.dot(st, wt_ref[...], preferred_element_type=jnp.float32)
        ai = ri * jnp.dot(si, wi_ref[...], preferred_element_type=jnp.float32)
        h = jax.nn.relu(at + ct * bt_ref[...] + ai + zi * bi_ref[...])
        hlo_ref[...] = h[:, :16]
        hhi_ref[...] = h[:, 16:]
        aux_ref[...] = jnp.concatenate(
            [ct, ri, zi, jnp.zeros((BLK, 5), jnp.float32)], axis=1)

    full = pl.BlockSpec((NC, BLK, LANES), lambda i: (0, i, 0))
    half = pl.BlockSpec((BLK, LANES), lambda i: (i, 0))
    w16 = pl.BlockSpec((16, 32), lambda i: (0, 0))
    b32 = pl.BlockSpec((1, 32), lambda i: (0, 0))
    return pl.pallas_call(
        body,
        grid=(grid,),
        in_specs=[full, full, w16, w16, b32, b32],
        out_specs=[half, half, pl.BlockSpec((BLK, 8), lambda i: (i, 0))],
        out_shape=[jax.ShapeDtypeStruct((npad, LANES), jnp.float32),
                   jax.ShapeDtypeStruct((npad, LANES), jnp.float32),
                   jax.ShapeDtypeStruct((npad, 8), jnp.float32)],
    )(St, Si, Wt0p, Wi0p, bt0, bi0)


def _tc_mid(St, Si, hlo, hhi, aux, Wt, bt, Wi, bi, npad):
    grid = npad // BLK

    def body(st_ref, si_ref, hlo_ref, hhi_ref, aux_ref,
             wt_ref, bt_ref, wi_ref, bi_ref, olo_ref, ohi_ref):
        wt = wt_ref[...]
        wi = wi_ref[...]
        at = (jnp.dot(st_ref[0], wt[:16], preferred_element_type=jnp.float32)
              + jnp.dot(st_ref[1], wt[16:], preferred_element_type=jnp.float32))
        al = (jnp.dot(si_ref[0], wi[:16], preferred_element_type=jnp.float32)
              + jnp.dot(si_ref[1], wi[16:], preferred_element_type=jnp.float32))
        ct = aux_ref[:, 0:1]
        ri = aux_ref[:, 1:2]
        zi = aux_ref[:, 2:3]
        conv = at + ct * bt_ref[...] + ri * al + zi * bi_ref[...]
        h = jax.nn.relu(conv)
        olo_ref[...] = h[:, :16] + hlo_ref[...]
        ohi_ref[...] = h[:, 16:] + hhi_ref[...]

    full = pl.BlockSpec((NC, BLK, LANES), lambda i: (0, i, 0))
    half = pl.BlockSpec((BLK, LANES), lambda i: (i, 0))
    aux_s = pl.BlockSpec((BLK, 8), lambda i: (i, 0))
    w32 = pl.BlockSpec((32, 32), lambda i: (0, 0))
    b32 = pl.BlockSpec((1, 32), lambda i: (0, 0))
    return pl.pallas_call(
        body,
        grid=(grid,),
        in_specs=[full, full, half, half, aux_s, w32, b32, w32, b32],
        out_specs=[half, half],
        out_shape=[jax.ShapeDtypeStruct((npad, LANES), jnp.float32),
                   jax.ShapeDtypeStruct((npad, LANES), jnp.float32)],
    )(St, Si, hlo, hhi, aux, Wt, bt, Wi, bi)


def _tc_final(St, Si, hlo, hhi, aux, Wt4, bt4, Wi4, bi4, Wproj, Wm, bm, npad):
    grid = npad // BLK

    def body(st_ref, si_ref, hlo_ref, hhi_ref, aux_ref, wt_ref, bt_ref,
             wi_ref, bi_ref, wp_ref, wm_ref, bm_ref, out_ref):
        wt = wt_ref[...]
        wi = wi_ref[...]
        wp = wp_ref[...]
        at = (jnp.dot(st_ref[0], wt[:16], preferred_element_type=jnp.float32)
              + jnp.dot(st_ref[1], wt[16:], preferred_element_type=jnp.float32))
        al = (jnp.dot(si_ref[0], wi[:16], preferred_element_type=jnp.float32)
              + jnp.dot(si_ref[1], wi[16:], preferred_element_type=jnp.float32))
        ct = aux_ref[:, 0:1]
        ri = aux_ref[:, 1:2]
        zi = aux_ref[:, 2:3]
        conv = at + ct * bt_ref[...] + ri * al + zi * bi_ref[...]
        z = jax.nn.relu(conv)
        skip = (jnp.dot(hlo_ref[...], wp[:16], preferred_element_type=jnp.float32)
                + jnp.dot(hhi_ref[...], wp[16:], preferred_element_type=jnp.float32))
        h4 = z + skip
        out_ref[...] = (jnp.dot(h4, wm_ref[...],
                                preferred_element_type=jnp.float32)
                        + bm_ref[...])

    full = pl.BlockSpec((NC, BLK, LANES), lambda i: (0, i, 0))
    half = pl.BlockSpec((BLK, LANES), lambda i: (i, 0))
    aux_s = pl.BlockSpec((BLK, 8), lambda i: (i, 0))
    w64 = pl.BlockSpec((32, 64), lambda i: (0, 0))
    b64 = pl.BlockSpec((1, 64), lambda i: (0, 0))
    wm_s = pl.BlockSpec((64, 32), lambda i: (0, 0))
    b32 = pl.BlockSpec((1, 32), lambda i: (0, 0))
    return pl.pallas_call(
        body,
        grid=(grid,),
        in_specs=[full, full, half, half, aux_s, w64, b64, w64, b64,
                  w64, wm_s, b32],
        out_specs=pl.BlockSpec((BLK, 32), lambda i: (i, 0)),
        out_shape=jax.ShapeDtypeStruct((npad, 32), jnp.float32),
    )(St, Si, hlo, hhi, aux, Wt4, bt4, Wi4, bi4, Wproj, Wm, bm)


def kernel(x, Wt0, bt0, Wi0, bi0, Wt1, bt1, Wi1, bi1, Wt2, bt2, Wi2, bi2,
           Wt3, bt3, Wi3, bi3, Wt4, bt4, Wi4, bi4, Wproj, Wm, bm,
           edge_index_temp, edge_index_intersects):
    n = x.shape[0]
    e = edge_index_temp.shape[1]
    npad, rows_per_tile, cpt, e_pad = _geometry(n, e)
    dump = n

    def prep(ei):
        src = jnp.concatenate(
            [ei[0], jnp.zeros((e_pad - e,), jnp.int32)])
        dst = jnp.concatenate(
            [ei[1], jnp.full((e_pad - e,), dump, jnp.int32)])
        return (src.reshape(e_pad // (GSZ * CHUNK), GSZ * CHUNK),
                dst.reshape(e_pad // (GSZ * CHUNK), GSZ * CHUNK))

    ts_src, ts_dst = prep(edge_index_temp)
    is_src, is_dst = prep(edge_index_intersects)

    x_aug = jnp.zeros((npad, LANES), jnp.float32)
    x_aug = x_aug.at[:n, :6].set(x)
    x_aug = x_aug.at[:n, 6].set(1.0)

    Wt0p = jnp.zeros((LANES, 32), jnp.float32).at[:6].set(Wt0)
    Wi0p = jnp.zeros((LANES, 32), jnp.float32).at[:6].set(Wi0)

    sc_l0 = _make_sc_layer(False, npad, rows_per_tile, cpt)
    sc_gen = _make_sc_layer(True, npad, rows_per_tile, cpt)

    St0, Si0 = sc_l0(x_aug, ts_src, ts_dst, is_src, is_dst)
    hlo, hhi, aux = _tc_layer0(St0, Si0, Wt0p, Wi0p,
                               bt0.reshape(1, -1), bi0.reshape(1, -1), npad)

    for Wt, bt, Wi, bi in ((Wt1, bt1, Wi1, bi1), (Wt2, bt2, Wi2, bi2),
                           (Wt3, bt3, Wi3, bi3)):
        St, Si = sc_gen(hlo, hhi, ts_src, ts_dst, is_src, is_dst)
        hlo, hhi = _tc_mid(St, Si, hlo, hhi, aux, Wt, bt.reshape(1, -1),
                           Wi, bi.reshape(1, -1), npad)

    St4, Si4 = sc_gen(hlo, hhi, ts_src, ts_dst, is_src, is_dst)
    out = _tc_final(St4, Si4, hlo, hhi, aux, Wt4, bt4.reshape(1, -1),
                    Wi4, bi4.reshape(1, -1), Wproj, Wm, bm.reshape(1, -1),
                    npad)
    return out[:n]
